# baseline (speedup 1.0000x reference)
import functools

import jax
import jax.numpy as jnp
from jax import lax
from jax.experimental import pallas as pl
from jax.experimental.pallas import tpu as pltpu
from jax.experimental.pallas import tpu_sc as plsc

N = 16384
K = 8192
D = 256
BR = 256
CHUNK = 1024
COMMIT = 0.25

INT_BIG = 2**31 - 1


def _argmin_body(x_ref, x2_ref, w2_ref, w_ref, idx_ref, loss_ref, acc_ref):
    x_blk = x_ref[...]
    x2_blk = x2_ref[...]

    def step(j, carry):
        run_min, run_idx = carry
        w_blk = w_ref[pl.ds(j * CHUNK, CHUNK), :]
        w2_blk = w2_ref[0, pl.ds(j * CHUNK, CHUNK)]
        mm = lax.dot_general(
            x_blk, w_blk,
            dimension_numbers=(((1,), (1,)), ((), ())),
            preferred_element_type=jnp.float32,
        )
        d2 = (x2_blk + w2_blk[None, :]) - 2.0 * mm
        dist = jnp.sqrt(jnp.maximum(d2, 0.0))
        bmin = jnp.min(dist, axis=1, keepdims=True)
        gidx = lax.broadcasted_iota(jnp.int32, (BR, CHUNK), 1) + j * CHUNK
        bidx = jnp.min(
            jnp.where(dist == bmin, gidx, INT_BIG), axis=1, keepdims=True
        )
        upd = bmin < run_min
        return (jnp.where(upd, bmin, run_min), jnp.where(upd, bidx, run_idx))

    init = (
        jnp.full((BR, 1), jnp.inf, dtype=jnp.float32),
        jnp.zeros((BR, 1), dtype=jnp.int32),
    )
    run_min, run_idx = lax.fori_loop(0, K // CHUNK, step, init)
    idx_ref[...] = run_idx

    part = jnp.sum(run_min * run_min)
    i = pl.program_id(0)
    prev = jnp.where(i == 0, 0.0, acc_ref[0])
    total = prev + part
    acc_ref[0] = total
    m = total / jnp.float32(N * D)
    loss_ref[...] = jnp.reshape(m + COMMIT * m, (1, 1))


def _argmin_call(flat, x2, w2, W):
    return pl.pallas_call(
        _argmin_body,
        grid=(N // BR,),
        in_specs=[
            pl.BlockSpec((BR, D), lambda i: (i, 0)),
            pl.BlockSpec((BR, 1), lambda i: (i, 0)),
            pl.BlockSpec((1, K), lambda i: (0, 0)),
            pl.BlockSpec((K, D), lambda i: (0, 0)),
        ],
        out_specs=[
            pl.BlockSpec((BR, 1), lambda i: (i, 0)),
            pl.BlockSpec((1, 1), lambda i: (0, 0)),
        ],
        out_shape=[
            jax.ShapeDtypeStruct((N, 1), jnp.int32),
            jax.ShapeDtypeStruct((1, 1), jnp.float32),
        ],
        scratch_shapes=[pltpu.SMEM((1,), jnp.float32)],
        compiler_params=pltpu.CompilerParams(
            dimension_semantics=("arbitrary",),
        ),
    )(flat, x2, w2, W)


ROWS_PER_WORKER = N // 32
GCHUNK = 128


def _gather_body(w_hbm, idx_hbm, out_hbm, idx_v, rows_v, sem):
    wid = lax.axis_index("s") * 2 + lax.axis_index("c")
    for c in range(ROWS_PER_WORKER // GCHUNK):
        base = wid * ROWS_PER_WORKER + c * GCHUNK
        pltpu.sync_copy(idx_hbm.at[pl.ds(base, GCHUNK)], idx_v)
        pltpu.async_copy(w_hbm.at[idx_v], rows_v, sem).wait()
        pltpu.sync_copy(rows_v, out_hbm.at[pl.ds(base, GCHUNK)])


def _gather_call(W, idx):
    mesh = plsc.VectorSubcoreMesh(core_axis_name="c", subcore_axis_name="s")
    f = functools.partial(
        pl.kernel,
        mesh=mesh,
        out_type=jax.ShapeDtypeStruct((N, D), jnp.float32),
        scratch_types=[
            pltpu.VMEM((GCHUNK,), jnp.int32),
            pltpu.VMEM((GCHUNK, D), jnp.float32),
            pltpu.SemaphoreType.DMA,
        ],
    )(_gather_body)
    return f(W, idx)


def kernel(inputs, W):
    flat = inputs.reshape(-1, D)
    x2 = jnp.sum(flat * flat, axis=1, keepdims=True)
    w2 = jnp.sum(W * W, axis=1)
    idx2d, loss2d = _argmin_call(flat, x2, w2.reshape(1, K), W)
    idx = idx2d.reshape(N)
    quantized = _gather_call(W, idx)
    loss = loss2d.reshape(())
    return (quantized, loss, idx)


# speedup vs baseline: 1.1941x; 1.1941x over previous
import functools

import jax
import jax.numpy as jnp
from jax import lax
from jax.experimental import pallas as pl
from jax.experimental.pallas import tpu as pltpu
from jax.experimental.pallas import tpu_sc as plsc

N = 16384
K = 8192
D = 256
BR = 256
CHUNK = 1024
COMMIT = 0.25

INT_BIG = 2**31 - 1


def _argmin_body(x_ref, x2_ref, w2_ref, w_ref, idx_ref, loss_ref, acc_ref):
    x_blk = x_ref[...]
    x2_blk = x2_ref[...]

    liota = lax.broadcasted_iota(jnp.int32, (BR, CHUNK), 1)

    run_min = jnp.full((BR, 1), jnp.inf, dtype=jnp.float32)
    run_idx = jnp.zeros((BR, 1), dtype=jnp.int32)
    for j in range(K // CHUNK):
        w_blk = w_ref[pl.ds(j * CHUNK, CHUNK), :]
        w2_blk = w2_ref[0, pl.ds(j * CHUNK, CHUNK)]
        mm = lax.dot_general(
            x_blk, w_blk,
            dimension_numbers=(((1,), (1,)), ((), ())),
            preferred_element_type=jnp.float32,
        )
        d2 = (x2_blk + w2_blk[None, :]) - 2.0 * mm
        dist = jnp.sqrt(jnp.maximum(d2, 0.0))
        bmin = jnp.min(dist, axis=1, keepdims=True)
        bidx = jnp.min(
            jnp.where(dist == bmin, liota, INT_BIG), axis=1, keepdims=True
        ) + j * CHUNK
        upd = bmin < run_min
        run_min = jnp.where(upd, bmin, run_min)
        run_idx = jnp.where(upd, bidx, run_idx)
    idx_ref[...] = run_idx

    part = jnp.sum(run_min * run_min)
    i = pl.program_id(0)
    prev = jnp.where(i == 0, 0.0, acc_ref[0])
    total = prev + part
    acc_ref[0] = total
    m = total / jnp.float32(N * D)
    loss_ref[...] = jnp.reshape(m + COMMIT * m, (1, 1))


def _argmin_call(flat, x2, w2, W):
    return pl.pallas_call(
        _argmin_body,
        grid=(N // BR,),
        in_specs=[
            pl.BlockSpec((BR, D), lambda i: (i, 0)),
            pl.BlockSpec((BR, 1), lambda i: (i, 0)),
            pl.BlockSpec((1, K), lambda i: (0, 0)),
            pl.BlockSpec((K, D), lambda i: (0, 0)),
        ],
        out_specs=[
            pl.BlockSpec((BR, 1), lambda i: (i, 0)),
            pl.BlockSpec((1, 1), lambda i: (0, 0)),
        ],
        out_shape=[
            jax.ShapeDtypeStruct((N, 1), jnp.int32),
            jax.ShapeDtypeStruct((1, 1), jnp.float32),
        ],
        scratch_shapes=[pltpu.SMEM((1,), jnp.float32)],
        compiler_params=pltpu.CompilerParams(
            dimension_semantics=("arbitrary",),
        ),
    )(flat, x2, w2, W)


ROWS_PER_WORKER = N // 32
GCHUNK = 128


def _gather_body(w_hbm, idx_hbm, out_hbm, idx_v, rows_v, sem):
    wid = lax.axis_index("s") * 2 + lax.axis_index("c")
    for c in range(ROWS_PER_WORKER // GCHUNK):
        base = wid * ROWS_PER_WORKER + c * GCHUNK
        pltpu.sync_copy(idx_hbm.at[pl.ds(base, GCHUNK)], idx_v)
        pltpu.async_copy(w_hbm.at[idx_v], rows_v, sem).wait()
        pltpu.sync_copy(rows_v, out_hbm.at[pl.ds(base, GCHUNK)])


def _gather_call(W, idx):
    mesh = plsc.VectorSubcoreMesh(core_axis_name="c", subcore_axis_name="s")
    f = functools.partial(
        pl.kernel,
        mesh=mesh,
        out_type=jax.ShapeDtypeStruct((N, D), jnp.float32),
        scratch_types=[
            pltpu.VMEM((GCHUNK,), jnp.int32),
            pltpu.VMEM((GCHUNK, D), jnp.float32),
            pltpu.SemaphoreType.DMA,
        ],
    )(_gather_body)
    return f(W, idx)


def kernel(inputs, W):
    flat = inputs.reshape(-1, D)
    x2 = jnp.sum(flat * flat, axis=1, keepdims=True)
    w2 = jnp.sum(W * W, axis=1)
    idx2d, loss2d = _argmin_call(flat, x2, w2.reshape(1, K), W)
    idx = idx2d.reshape(N)
    quantized = _gather_call(W, idx)
    loss = loss2d.reshape(())
    return (quantized, loss, idx)


# speedup vs baseline: 1.2624x; 1.0572x over previous
import functools

import jax
import jax.numpy as jnp
from jax import lax
from jax.experimental import pallas as pl
from jax.experimental.pallas import tpu as pltpu
from jax.experimental.pallas import tpu_sc as plsc

N = 16384
K = 8192
D = 256
BR = 128
CHUNK = 1024
COMMIT = 0.25

FLT_BIG = 1e30


def _argmin_body(x_ref, x2_ref, w2_ref, w_ref, idx_ref, loss_ref,
                 d2_ref, acc_ref):
    x_blk = x_ref[...]
    x2_blk = x2_ref[...]

    NV = CHUNK // 128
    liota128_f = lax.broadcasted_iota(jnp.int32, (BR, 128), 1).astype(jnp.float32)

    run128 = None
    for j in range(K // CHUNK):
        w_blk = w_ref[pl.ds(j * CHUNK, CHUNK), :]
        w2_blk = w2_ref[0, pl.ds(j * CHUNK, CHUNK)]
        mm2 = lax.dot_general(
            x_blk, w_blk,
            dimension_numbers=(((1,), (1,)), ((), ())),
            preferred_element_type=jnp.float32,
        )
        d2 = (x2_blk + w2_blk[None, :]) - mm2
        dist = jnp.sqrt(jnp.maximum(d2, 0.0))
        d2_ref[:, pl.ds(j * CHUNK, CHUNK)] = dist
        fold = dist[:, 0:128]
        for v in range(1, NV):
            fold = jnp.minimum(fold, dist[:, v * 128:(v + 1) * 128])
        run128 = fold if run128 is None else jnp.minimum(run128, fold)

    s = jnp.min(run128, axis=1, keepdims=True)

    run_enc = None
    for j in range(K // CHUNK):
        dist = d2_ref[:, pl.ds(j * CHUNK, CHUNK)]
        enc128 = None
        for v in range(NV):
            ev = jnp.where(dist[:, v * 128:(v + 1) * 128] <= s,
                           jnp.float32(j * CHUNK + v * 128), FLT_BIG)
            enc128 = ev if enc128 is None else jnp.minimum(enc128, ev)
        run_enc = enc128 if run_enc is None else jnp.minimum(run_enc, enc128)

    run_idx_f = jnp.min(run_enc + liota128_f, axis=1, keepdims=True)
    run_min = s * s
    idx_ref[...] = run_idx_f

    part = jnp.sum(run_min)
    i = pl.program_id(0)
    prev = jnp.where(i == 0, 0.0, acc_ref[0])
    total = prev + part
    acc_ref[0] = total
    m = total / jnp.float32(N * D)
    loss_ref[...] = jnp.reshape(m + COMMIT * m, (1, 1))


def _argmin_call(flat, x2, w2, W):
    return pl.pallas_call(
        _argmin_body,
        grid=(N // BR,),
        in_specs=[
            pl.BlockSpec((BR, D), lambda i: (i, 0)),
            pl.BlockSpec((BR, 1), lambda i: (i, 0)),
            pl.BlockSpec((1, K), lambda i: (0, 0)),
            pl.BlockSpec((K, D), lambda i: (0, 0)),
        ],
        out_specs=[
            pl.BlockSpec((BR, 1), lambda i: (i, 0)),
            pl.BlockSpec((1, 1), lambda i: (0, 0)),
        ],
        out_shape=[
            jax.ShapeDtypeStruct((N, 1), jnp.float32),
            jax.ShapeDtypeStruct((1, 1), jnp.float32),
        ],
        scratch_shapes=[
            pltpu.VMEM((BR, K), jnp.float32),
            pltpu.SMEM((1,), jnp.float32),
        ],
        compiler_params=pltpu.CompilerParams(
            dimension_semantics=("arbitrary",),
        ),
    )(flat, x2, w2, W)


ROWS_PER_WORKER = N // 32
GCHUNK = 128


def _gather_body(w_hbm, idx_hbm, out_hbm, idx_v, rows_v, sem):
    wid = lax.axis_index("s") * 2 + lax.axis_index("c")
    for c in range(ROWS_PER_WORKER // GCHUNK):
        base = wid * ROWS_PER_WORKER + c * GCHUNK
        pltpu.sync_copy(idx_hbm.at[pl.ds(base, GCHUNK)], idx_v)
        pltpu.async_copy(w_hbm.at[idx_v], rows_v, sem).wait()
        pltpu.sync_copy(rows_v, out_hbm.at[pl.ds(base, GCHUNK)])


def _gather_call(W, idx):
    mesh = plsc.VectorSubcoreMesh(core_axis_name="c", subcore_axis_name="s")
    f = functools.partial(
        pl.kernel,
        mesh=mesh,
        out_type=jax.ShapeDtypeStruct((N, D), jnp.float32),
        scratch_types=[
            pltpu.VMEM((GCHUNK,), jnp.int32),
            pltpu.VMEM((GCHUNK, D), jnp.float32),
            pltpu.SemaphoreType.DMA,
        ],
    )(_gather_body)
    return f(W, idx)


def kernel(inputs, W):
    flat = inputs.reshape(-1, D)
    x2 = jnp.sum(flat * flat, axis=1, keepdims=True)
    w2 = jnp.sum(W * W, axis=1)
    idx2d, loss2d = _argmin_call(2.0 * flat, x2, w2.reshape(1, K), W)
    idx = idx2d.reshape(N).astype(jnp.int32)
    quantized = _gather_call(W, idx)
    loss = loss2d.reshape(())
    return (quantized, loss, idx)


# speedup vs baseline: 1.4524x; 1.1506x over previous
import functools

import jax
import jax.numpy as jnp
from jax import lax
from jax.experimental import pallas as pl
from jax.experimental.pallas import tpu as pltpu
from jax.experimental.pallas import tpu_sc as plsc

N = 16384
K = 8192
D = 256
BR = 256
CHUNK = 1024
COMMIT = 0.25

FLT_BIG = 1e30


def _argmin_body(x_ref, x2_ref, w2_ref, w_ref, idx_ref, loss_ref,
                 d2_ref, acc_ref):
    x_blk = x_ref[...]
    x2_blk = x2_ref[...]

    NV = CHUNK // 128
    liota128_f = lax.broadcasted_iota(jnp.int32, (BR, 128), 1).astype(jnp.float32)

    run128 = None
    for j in range(K // CHUNK):
        w_blk = w_ref[pl.ds(j * CHUNK, CHUNK), :]
        w2_blk = w2_ref[0, pl.ds(j * CHUNK, CHUNK)]
        mm2 = lax.dot_general(
            x_blk, w_blk,
            dimension_numbers=(((1,), (1,)), ((), ())),
            preferred_element_type=jnp.float32,
        )
        d2 = (x2_blk + w2_blk[None, :]) - mm2
        dist = jnp.sqrt(jnp.maximum(d2, 0.0))
        d2_ref[:, pl.ds(j * CHUNK, CHUNK)] = dist
        fold = dist[:, 0:128]
        for v in range(1, NV):
            fold = jnp.minimum(fold, dist[:, v * 128:(v + 1) * 128])
        run128 = fold if run128 is None else jnp.minimum(run128, fold)

    s = jnp.min(run128, axis=1, keepdims=True)

    run_enc = None
    for j in range(K // CHUNK):
        dist = d2_ref[:, pl.ds(j * CHUNK, CHUNK)]
        enc128 = None
        for v in range(NV):
            ev = jnp.where(dist[:, v * 128:(v + 1) * 128] <= s,
                           jnp.float32(j * CHUNK + v * 128), FLT_BIG)
            enc128 = ev if enc128 is None else jnp.minimum(enc128, ev)
        run_enc = enc128 if run_enc is None else jnp.minimum(run_enc, enc128)

    run_idx_f = jnp.min(run_enc + liota128_f, axis=1, keepdims=True)
    run_min = s * s
    idx_ref[...] = run_idx_f

    part = jnp.sum(run_min)
    i = pl.program_id(0)
    prev = jnp.where(i == 0, 0.0, acc_ref[0])
    total = prev + part
    acc_ref[0] = total
    m = total / jnp.float32(N * D)
    loss_ref[...] = jnp.reshape(m + COMMIT * m, (1, 1))


def _argmin_call(flat, x2, w2, W):
    return pl.pallas_call(
        _argmin_body,
        grid=(N // BR,),
        in_specs=[
            pl.BlockSpec((BR, D), lambda i: (i, 0)),
            pl.BlockSpec((BR, 1), lambda i: (i, 0)),
            pl.BlockSpec((1, K), lambda i: (0, 0)),
            pl.BlockSpec((K, D), lambda i: (0, 0)),
        ],
        out_specs=[
            pl.BlockSpec((BR, 1), lambda i: (i, 0)),
            pl.BlockSpec((1, 1), lambda i: (0, 0)),
        ],
        out_shape=[
            jax.ShapeDtypeStruct((N, 1), jnp.float32),
            jax.ShapeDtypeStruct((1, 1), jnp.float32),
        ],
        scratch_shapes=[
            pltpu.VMEM((BR, K), jnp.float32),
            pltpu.SMEM((1,), jnp.float32),
        ],
        compiler_params=pltpu.CompilerParams(
            dimension_semantics=("arbitrary",),
        ),
    )(flat, x2, w2, W)


ROWS_PER_WORKER = N // 32
GCHUNK = 128


def _gather_body(w_hbm, idx_hbm, out_hbm, idx_v, rows_v, sem):
    wid = lax.axis_index("s") * 2 + lax.axis_index("c")
    for c in range(ROWS_PER_WORKER // GCHUNK):
        base = wid * ROWS_PER_WORKER + c * GCHUNK
        pltpu.sync_copy(idx_hbm.at[pl.ds(base, GCHUNK)], idx_v)
        pltpu.async_copy(w_hbm.at[idx_v], rows_v, sem).wait()
        pltpu.sync_copy(rows_v, out_hbm.at[pl.ds(base, GCHUNK)])


def _gather_call(W, idx):
    mesh = plsc.VectorSubcoreMesh(core_axis_name="c", subcore_axis_name="s")
    f = functools.partial(
        pl.kernel,
        mesh=mesh,
        out_type=jax.ShapeDtypeStruct((N, D), jnp.float32),
        scratch_types=[
            pltpu.VMEM((GCHUNK,), jnp.int32),
            pltpu.VMEM((GCHUNK, D), jnp.float32),
            pltpu.SemaphoreType.DMA,
        ],
    )(_gather_body)
    return f(W, idx)


def kernel(inputs, W):
    flat = inputs.reshape(-1, D)
    x2 = jnp.sum(flat * flat, axis=1, keepdims=True)
    w2 = jnp.sum(W * W, axis=1)
    idx2d, loss2d = _argmin_call(2.0 * flat, x2, w2.reshape(1, K), W)
    idx = idx2d.reshape(N).astype(jnp.int32)
    quantized = _gather_call(W, idx)
    loss = loss2d.reshape(())
    return (quantized, loss, idx)
